# baseline (speedup 1.0000x reference)
import functools

import jax
import jax.numpy as jnp
from jax import lax
from jax.experimental import pallas as pl
from jax.experimental.pallas import tpu as pltpu
from jax.experimental.pallas import tpu_sc as plsc

_NC = 2
_NS = 16
_NW = _NC * _NS
_LANES = 16
_CH = 128


@functools.lru_cache(maxsize=None)
def _build(BN, V, D, N):
    rows_per_w = BN // _NW
    n_ch = rows_per_w // _CH
    vregs_per_row = D // _LANES
    mesh = plsc.VectorSubcoreMesh(core_axis_name="c", subcore_axis_name="s")

    @functools.partial(
        pl.kernel,
        mesh=mesh,
        out_type=jax.ShapeDtypeStruct((BN, D), jnp.float32),
        scratch_types=[
            pltpu.VMEM((_CH,), jnp.int32),
            pltpu.VMEM((_CH, D), jnp.float32),
            pltpu.VMEM((N + _CH, D), jnp.float32),
            pltpu.SemaphoreType.DMA,
        ],
        compiler_params=pltpu.CompilerParams(use_tc_tiling_on_sc=False),
    )
    def emb(idx_hbm, tok_hbm, pos_hbm, out_hbm, idx_v, rows_v, pos_v, sem):
        wid = lax.axis_index("s") * _NC + lax.axis_index("c")
        base = wid * rows_per_w
        pltpu.sync_copy(pos_hbm, pos_v)

        def chunk_body(c, carry):
            rbase = base + c * _CH
            pltpu.sync_copy(idx_hbm.at[pl.ds(rbase, _CH)], idx_v)
            pltpu.async_copy(tok_hbm.at[idx_v], rows_v, sem).wait()
            o_c = lax.rem(c * _CH, N)

            def add_row(r, carry2):
                p = o_c + r
                for v in range(vregs_per_row):
                    sl = pl.ds(v * _LANES, _LANES)
                    rows_v[r, sl] = rows_v[r, sl] + pos_v[p, sl]
                return carry2

            lax.fori_loop(0, _CH, add_row, 0)
            pltpu.sync_copy(rows_v, out_hbm.at[pl.ds(rbase, _CH)])
            return carry

        lax.fori_loop(0, n_ch, chunk_body, 0)

    return emb


def kernel(indices, tok_emb_table, pos_emb_table):
    B, N = indices.shape
    V, D = tok_emb_table.shape
    idx_flat = indices.reshape(B * N).astype(jnp.int32)
    pos = pos_emb_table[:N].astype(jnp.float32)
    pos_ext = jnp.concatenate([pos, pos[:_CH]], axis=0)
    out = _build(B * N, V, D, N)(idx_flat, tok_emb_table, pos_ext)
    return out.reshape(B, N, D)


# speedup vs baseline: 1.4382x; 1.4382x over previous
import functools

import jax
import jax.numpy as jnp
from jax import lax
from jax.experimental import pallas as pl
from jax.experimental.pallas import tpu as pltpu
from jax.experimental.pallas import tpu_sc as plsc

_NC = 2
_NS = 16
_NW = _NC * _NS
_CH = 128


@functools.lru_cache(maxsize=None)
def _build(BN, V, D, N):
    rows_per_w = BN // _NW
    n_ch = rows_per_w // _CH
    n_pairs = n_ch // 2
    mesh = plsc.VectorSubcoreMesh(core_axis_name="c", subcore_axis_name="s")

    @functools.partial(
        pl.kernel,
        mesh=mesh,
        out_type=jax.ShapeDtypeStruct((BN, D), jnp.float32),
        scratch_types=[
            pltpu.VMEM((n_ch, _CH), jnp.int32),
            pltpu.VMEM((_CH, D), jnp.float32),
            pltpu.VMEM((_CH, D), jnp.float32),
            pltpu.VMEM_SHARED((N + _CH, D), jnp.float32),
            pltpu.SemaphoreType.DMA,
            pltpu.SemaphoreType.DMA,
            pltpu.SemaphoreType.DMA,
            pltpu.SemaphoreType.DMA,
        ],
        compiler_params=pltpu.CompilerParams(use_tc_tiling_on_sc=False),
    )
    def emb(idx_hbm, tok_hbm, pos_hbm, out_hbm,
            idx_all, rows0, rows1, pos_sh, g0, g1, o0, o1):
        cid = lax.axis_index("c")
        sid = lax.axis_index("s")
        wid = sid * _NC + cid
        base = wid * rows_per_w
        rows = (rows0, rows1)
        gsem = (g0, g1)
        osem = (o0, o1)

        @pl.when(sid == 0)
        def _():
            pltpu.sync_copy(pos_hbm, pos_sh)

        plsc.subcore_barrier()

        pltpu.sync_copy(idx_hbm.at[pl.ds(wid * n_ch, n_ch)], idx_all)

        def prefill_and_gather(c, b):
            o_c = lax.rem(c * _CH, N)
            pltpu.sync_copy(pos_sh.at[pl.ds(o_c, _CH)], rows[b])
            pltpu.async_copy(tok_hbm.at[idx_all.at[c]], rows[b], gsem[b],
                             add=True)

        def wait_gather(c, b):
            pltpu.make_async_copy(tok_hbm.at[idx_all.at[c]], rows[b],
                                  gsem[b]).wait()

        def issue_writeout(c, b):
            pltpu.async_copy(rows[b], out_hbm.at[pl.ds(base + c * _CH, _CH)],
                             osem[b])

        def wait_writeout(b):
            pltpu.make_async_copy(rows[b], out_hbm.at[pl.ds(base, _CH)],
                                  osem[b]).wait()

        prefill_and_gather(0, 0)

        def pair_body(j, carry):
            c = 2 * j
            wait_gather(c, 0)
            issue_writeout(c, 0)

            @pl.when(j > 0)
            def _():
                wait_writeout(1)

            prefill_and_gather(c + 1, 1)

            wait_gather(c + 1, 1)
            issue_writeout(c + 1, 1)
            wait_writeout(0)

            @pl.when(j < n_pairs - 1)
            def _():
                prefill_and_gather(c + 2, 0)

            return carry

        lax.fori_loop(0, n_pairs, pair_body, 0)
        wait_writeout(1)

    return emb


def kernel(indices, tok_emb_table, pos_emb_table):
    B, N = indices.shape
    V, D = tok_emb_table.shape
    BN = B * N
    idx2d = indices.reshape(BN // _CH, _CH).astype(jnp.int32)
    pos = pos_emb_table[:N].astype(jnp.float32)
    pos_ext = jnp.concatenate([pos, pos[:_CH]], axis=0)
    out = _build(BN, V, D, N)(idx2d, tok_emb_table, pos_ext)
    return out.reshape(B, N, D)
